# Initial kernel scaffold; baseline (speedup 1.0000x reference)
import jax
import jax.numpy as jnp
from jax.experimental import pallas as pl


def kernel(x, edge_index, W1, b1, g1, be1, m1, v1, W2, b2, g2, be2, m2, v2, W3, b3):
    raise NotImplementedError("write your pallas kernel here")


# speedup vs baseline: 12.8295x; 12.8295x over previous
import functools

import jax
import jax.numpy as jnp
from jax import lax
from jax.experimental import pallas as pl
from jax.experimental.pallas import tpu as pltpu
from jax.experimental.pallas import tpu_sc as plsc

NC = 2
NS = 16
LANE = 128

_MESH = plsc.VectorSubcoreMesh(core_axis_name="c", subcore_axis_name="s")


def _deg_body(rpw, rpt, dst_hbm, ones_hbm, zeros_hbm, out_hbm,
              acc, dst_v, ones_v, buf_v, sem):
    c = lax.axis_index("c")
    s = lax.axis_index("s")
    pltpu.sync_copy(zeros_hbm, buf_v)
    pltpu.sync_copy(buf_v, acc.at[pl.ds(s * rpt, rpt)])
    pltpu.sync_copy(ones_hbm, ones_v)
    plsc.subcore_barrier()
    base = (c * NS + s) * rpw

    def body(j, carry):
        pltpu.sync_copy(dst_hbm.at[base + j], dst_v)
        pltpu.sync_copy(ones_v, acc.at[dst_v], add=True)
        return carry

    lax.fori_loop(0, rpw, body, 0)
    plsc.subcore_barrier()
    pltpu.sync_copy(acc.at[pl.ds(s * rpt, rpt)], buf_v)
    pltpu.sync_copy(buf_v, out_hbm.at[c, pl.ds(s * rpt, rpt)])


def _agg_body(rpw, rpt, y_hbm, src_hbm, dst_hbm, zeros_hbm, out_hbm,
              acc, src_v, dst_v, rows_v, sem):
    c = lax.axis_index("c")
    s = lax.axis_index("s")
    pltpu.sync_copy(zeros_hbm, rows_v)
    for k in range(rpt // LANE):
        pltpu.sync_copy(rows_v, acc.at[pl.ds(s * rpt + k * LANE, LANE)])
    plsc.subcore_barrier()
    base = (c * NS + s) * rpw

    def body(j, carry):
        pltpu.sync_copy(src_hbm.at[base + j], src_v)
        pltpu.sync_copy(dst_hbm.at[base + j], dst_v)
        pltpu.async_copy(y_hbm.at[src_v], rows_v, sem).wait()
        pltpu.sync_copy(rows_v, acc.at[dst_v], add=True)
        return carry

    lax.fori_loop(0, rpw, body, 0)
    plsc.subcore_barrier()
    for k in range(rpt // LANE):
        r = s * rpt + k * LANE
        pltpu.sync_copy(acc.at[pl.ds(r, LANE)], rows_v)
        pltpu.sync_copy(rows_v, out_hbm.at[c, pl.ds(r, LANE)])


_SC_PARAMS = pltpu.CompilerParams(use_tc_tiling_on_sc=False)


def _deg_call(dst2d, npad, rpw, rpt):
    return pl.kernel(
        functools.partial(_deg_body, rpw, rpt),
        out_type=jax.ShapeDtypeStruct((NC, npad), jnp.float32),
        mesh=_MESH,
        compiler_params=_SC_PARAMS,
        scratch_types=[
            pltpu.VMEM_SHARED((npad,), jnp.float32),
            pltpu.VMEM((LANE,), jnp.int32),
            pltpu.VMEM((LANE,), jnp.float32),
            pltpu.VMEM((rpt,), jnp.float32),
            pltpu.SemaphoreType.DMA,
        ],
    )(dst2d, jnp.ones((LANE,), jnp.float32), jnp.zeros((rpt,), jnp.float32))


def _agg_call(y, src2d, dst2d, npad, rpw, rpt):
    f = y.shape[1]
    return pl.kernel(
        functools.partial(_agg_body, rpw, rpt),
        out_type=jax.ShapeDtypeStruct((NC, npad, f), jnp.float32),
        mesh=_MESH,
        compiler_params=_SC_PARAMS,
        scratch_types=[
            pltpu.VMEM_SHARED((npad, f), jnp.float32),
            pltpu.VMEM((LANE,), jnp.int32),
            pltpu.VMEM((LANE,), jnp.int32),
            pltpu.VMEM((LANE, f), jnp.float32),
            pltpu.SemaphoreType.DMA,
        ],
    )(y, src2d, dst2d, jnp.zeros((LANE, f), jnp.float32))


def _dot(a, b):
    return jax.lax.dot(a, b, precision=jax.lax.Precision.HIGHEST,
                       preferred_element_type=jnp.float32)


def _tc_first(x_ref, w_ref, da_ref, db_ref, o_ref):
    dinv = lax.rsqrt(da_ref[...] + db_ref[...] + 1.0)
    o_ref[...] = _dot(x_ref[...], w_ref[...]) * dinv


def _tc_mid(za_ref, zb_ref, y_ref, da_ref, db_ref, b_ref, g_ref, be_ref,
            m_ref, v_ref, w_ref, o_ref):
    dinv = lax.rsqrt(da_ref[...] + db_ref[...] + 1.0)
    t = dinv * (za_ref[...] + zb_ref[...] + y_ref[...]) + b_ref[...]
    t = g_ref[...] * (t - m_ref[...]) * lax.rsqrt(v_ref[...] + 1e-5) + be_ref[...]
    t = jnp.maximum(t, 0.0)
    o_ref[...] = _dot(t, w_ref[...]) * dinv


def _tc_last(za_ref, zb_ref, y_ref, da_ref, db_ref, b_ref, o_ref):
    dinv = lax.rsqrt(da_ref[...] + db_ref[...] + 1.0)
    logits = dinv * (za_ref[...] + zb_ref[...] + y_ref[...]) + b_ref[...]
    mx = jnp.max(logits, axis=1, keepdims=True)
    sh = logits - mx
    o_ref[...] = sh - jnp.log(jnp.sum(jnp.exp(sh), axis=1, keepdims=True))


def _row_spec(bm, f):
    return pl.BlockSpec((bm, f), lambda i: (i, 0))


def _full_spec(shape):
    return pl.BlockSpec(shape, lambda i: (0,) * len(shape))


def _tc_first_call(x, w, da, db, bm=2000):
    n, f_in = x.shape
    h = w.shape[1]
    return pl.pallas_call(
        _tc_first,
        grid=(n // bm,),
        in_specs=[_row_spec(bm, f_in), _full_spec(w.shape),
                  _row_spec(bm, 1), _row_spec(bm, 1)],
        out_specs=_row_spec(bm, h),
        out_shape=jax.ShapeDtypeStruct((n, h), jnp.float32),
    )(x, w, da, db)


def _tc_mid_call(za, zb, y, da, db, b, g, be, m, v, w, bm=2000):
    n, h = y.shape
    h2 = w.shape[1]
    row1 = lambda a: a.reshape(1, -1)
    return pl.pallas_call(
        _tc_mid,
        grid=(n // bm,),
        in_specs=[_row_spec(bm, h)] * 3 + [_row_spec(bm, 1)] * 2
                 + [_full_spec((1, h))] * 5 + [_full_spec(w.shape)],
        out_specs=_row_spec(bm, h2),
        out_shape=jax.ShapeDtypeStruct((n, h2), jnp.float32),
    )(za, zb, y, da, db, row1(b), row1(g), row1(be), row1(m), row1(v), w)


def _tc_last_call(za, zb, y, da, db, b, bm=2000):
    n, c = y.shape
    return pl.pallas_call(
        _tc_last,
        grid=(n // bm,),
        in_specs=[_row_spec(bm, c)] * 3 + [_row_spec(bm, 1)] * 2
                 + [_full_spec((1, c))],
        out_specs=_row_spec(bm, c),
        out_shape=jax.ShapeDtypeStruct((n, c), jnp.float32),
    )(za, zb, y, da, db, b.reshape(1, -1))


def kernel(x, edge_index, W1, b1, g1, be1, m1, v1, W2, b2, g2, be2, m2, v2,
           W3, b3):
    n = x.shape[0]
    e = edge_index.shape[1]
    nw = NC * NS
    rpt = -(-(n + 1) // (NS * LANE)) * LANE
    npad = NS * rpt
    rpw = -(-e // (nw * LANE))
    e_pad = nw * rpw * LANE
    src2d = jnp.concatenate(
        [edge_index[0], jnp.zeros((e_pad - e,), jnp.int32)]).reshape(-1, LANE)
    dst2d = jnp.concatenate(
        [edge_index[1], jnp.full((e_pad - e,), n, jnp.int32)]).reshape(-1, LANE)

    degp = _deg_call(dst2d, npad, rpw, rpt)
    da = degp[0, :n, None]
    db = degp[1, :n, None]

    y1 = _tc_first_call(x, W1, da, db)
    z1 = _agg_call(y1, src2d, dst2d, npad, rpw, rpt)
    y2 = _tc_mid_call(z1[0, :n], z1[1, :n], y1, da, db, b1, g1, be1, m1, v1, W2)
    z2 = _agg_call(y2, src2d, dst2d, npad, rpw, rpt)
    y3 = _tc_mid_call(z2[0, :n], z2[1, :n], y2, da, db, b2, g2, be2, m2, v2, W3)
    z3 = _agg_call(y3, src2d, dst2d, npad, rpw, rpt)
    return _tc_last_call(z3[0, :n], z3[1, :n], y3, da, db, b3)


# speedup vs baseline: 13.9009x; 1.0835x over previous
import functools

import jax
import jax.numpy as jnp
from jax import lax
from jax.experimental import pallas as pl
from jax.experimental.pallas import tpu as pltpu
from jax.experimental.pallas import tpu_sc as plsc

NC = 2
NS = 16
LANE = 128
NBUF = 4

_MESH = plsc.VectorSubcoreMesh(core_axis_name="c", subcore_axis_name="s")
_SC_PARAMS = pltpu.CompilerParams(use_tc_tiling_on_sc=False)


def _deg_body(rpw, rpt, nbuf, dst_hbm, ones_hbm, zeros_hbm, out_hbm,
              acc, dst_all, ones_v, buf_v, *sems):
    c = lax.axis_index("c")
    s = lax.axis_index("s")
    pltpu.sync_copy(zeros_hbm, buf_v)
    pltpu.sync_copy(buf_v, acc.at[pl.ds(s * rpt, rpt)])
    pltpu.sync_copy(ones_hbm, ones_v)
    plsc.subcore_barrier()
    base = (c * NS + s) * rpw
    pltpu.sync_copy(dst_hbm.at[pl.ds(base, rpw)], dst_all)
    ngroups = rpw // nbuf
    for b in range(nbuf):
        pltpu.async_copy(ones_v, acc.at[dst_all.at[b]], sems[b], add=True)

    def group(g, carry):
        for b in range(nbuf):
            pltpu.make_async_copy(ones_v, acc.at[dst_all.at[0]],
                                  sems[b]).wait()
            pltpu.async_copy(ones_v, acc.at[dst_all.at[(g + 1) * nbuf + b]],
                             sems[b], add=True)
        return carry

    lax.fori_loop(0, ngroups - 1, group, 0)
    for b in range(nbuf):
        pltpu.make_async_copy(ones_v, acc.at[dst_all.at[0]], sems[b]).wait()
    plsc.subcore_barrier()
    pltpu.sync_copy(acc.at[pl.ds(s * rpt, rpt)], buf_v)
    pltpu.sync_copy(buf_v, out_hbm.at[c, pl.ds(s * rpt, rpt)])


def _agg_body(rpw, rpt, nbuf, phases, y_hbm, src_hbm, dst_hbm, zeros_hbm,
              out_hbm, acc, src_all, dst_all, rows, *sems):
    g_sems = sems[:nbuf]
    s_sems = sems[nbuf:]
    c = lax.axis_index("c")
    s = lax.axis_index("s")
    gpre = rpw // phases
    ngroups = gpre // nbuf
    pltpu.sync_copy(zeros_hbm, rows.at[0])
    for k in range(rpt // LANE):
        pltpu.sync_copy(rows.at[0], acc.at[pl.ds(s * rpt + k * LANE, LANE)])
    plsc.subcore_barrier()
    base = (c * NS + s) * rpw

    def gather(row_in_phase, b):
        return pltpu.async_copy(y_hbm.at[src_all.at[row_in_phase]],
                                rows.at[b], g_sems[b])

    def scatter(row_in_phase, b):
        return pltpu.async_copy(rows.at[b], acc.at[dst_all.at[row_in_phase]],
                                s_sems[b], add=True)

    for ph in range(phases):
        pltpu.sync_copy(src_hbm.at[pl.ds(base + ph * gpre, gpre)], src_all)
        pltpu.sync_copy(dst_hbm.at[pl.ds(base + ph * gpre, gpre)], dst_all)
        for b in range(nbuf):
            gather(b, b)

        def group(g, carry):
            for b in range(nbuf):
                pltpu.make_async_copy(y_hbm.at[src_all.at[0]], rows.at[b],
                                      g_sems[b]).wait()
                scatter(g * nbuf + b, b)
            for b in range(nbuf):
                pltpu.make_async_copy(rows.at[b], acc.at[dst_all.at[0]],
                                      s_sems[b]).wait()
                gather((g + 1) * nbuf + b, b)
            return carry

        lax.fori_loop(0, ngroups - 1, group, 0)
        for b in range(nbuf):
            pltpu.make_async_copy(y_hbm.at[src_all.at[0]], rows.at[b],
                                  g_sems[b]).wait()
            scatter((ngroups - 1) * nbuf + b, b)
        for b in range(nbuf):
            pltpu.make_async_copy(rows.at[b], acc.at[dst_all.at[0]],
                                  s_sems[b]).wait()

    plsc.subcore_barrier()
    for k in range(rpt // LANE):
        r = s * rpt + k * LANE
        pltpu.sync_copy(acc.at[pl.ds(r, LANE)], rows.at[0])
        pltpu.sync_copy(rows.at[0], out_hbm.at[c, pl.ds(r, LANE)])


def _deg_call(dst2d, npad, rpw, rpt, nbuf=8):
    return pl.kernel(
        functools.partial(_deg_body, rpw, rpt, nbuf),
        out_type=jax.ShapeDtypeStruct((NC, npad), jnp.float32),
        mesh=_MESH,
        compiler_params=_SC_PARAMS,
        scratch_types=[
            pltpu.VMEM_SHARED((npad,), jnp.float32),
            pltpu.VMEM((rpw, LANE), jnp.int32),
            pltpu.VMEM((LANE,), jnp.float32),
            pltpu.VMEM((rpt,), jnp.float32),
        ] + [pltpu.SemaphoreType.DMA] * nbuf,
    )(dst2d, jnp.ones((LANE,), jnp.float32), jnp.zeros((rpt,), jnp.float32))


def _agg_call(y, src2d, dst2d, npad, rpw, rpt):
    f = y.shape[1]
    nbuf = max(2, min(8, 256 // f))
    phases = 2 if f >= 128 else 1
    return pl.kernel(
        functools.partial(_agg_body, rpw, rpt, nbuf, phases),
        out_type=jax.ShapeDtypeStruct((NC, npad, f), jnp.float32),
        mesh=_MESH,
        compiler_params=_SC_PARAMS,
        scratch_types=[
            pltpu.VMEM_SHARED((npad, f), jnp.float32),
            pltpu.VMEM((rpw // phases, LANE), jnp.int32),
            pltpu.VMEM((rpw // phases, LANE), jnp.int32),
            pltpu.VMEM((nbuf, LANE, f), jnp.float32),
        ] + [pltpu.SemaphoreType.DMA] * (2 * nbuf),
    )(y, src2d, dst2d, jnp.zeros((LANE, f), jnp.float32))


def _dot(a, b):
    return jax.lax.dot(a, b, precision=jax.lax.Precision.HIGHEST,
                       preferred_element_type=jnp.float32)


def _tc_first(x_ref, w_ref, da_ref, db_ref, o_ref):
    dinv = lax.rsqrt(da_ref[...] + db_ref[...] + 1.0)
    o_ref[...] = _dot(x_ref[...], w_ref[...]) * dinv


def _tc_mid(za_ref, zb_ref, y_ref, da_ref, db_ref, b_ref, g_ref, be_ref,
            m_ref, v_ref, w_ref, o_ref):
    dinv = lax.rsqrt(da_ref[...] + db_ref[...] + 1.0)
    t = dinv * (za_ref[...] + zb_ref[...] + y_ref[...]) + b_ref[...]
    t = g_ref[...] * (t - m_ref[...]) * lax.rsqrt(v_ref[...] + 1e-5) + be_ref[...]
    t = jnp.maximum(t, 0.0)
    o_ref[...] = _dot(t, w_ref[...]) * dinv


def _tc_last(za_ref, zb_ref, y_ref, da_ref, db_ref, b_ref, o_ref):
    dinv = lax.rsqrt(da_ref[...] + db_ref[...] + 1.0)
    logits = dinv * (za_ref[...] + zb_ref[...] + y_ref[...]) + b_ref[...]
    mx = jnp.max(logits, axis=1, keepdims=True)
    sh = logits - mx
    o_ref[...] = sh - jnp.log(jnp.sum(jnp.exp(sh), axis=1, keepdims=True))


def _row_spec(bm, f):
    return pl.BlockSpec((bm, f), lambda i: (i, 0))


def _full_spec(shape):
    return pl.BlockSpec(shape, lambda i: (0,) * len(shape))


def _tc_first_call(x, w, da, db, bm=2000):
    n, f_in = x.shape
    h = w.shape[1]
    return pl.pallas_call(
        _tc_first,
        grid=(n // bm,),
        in_specs=[_row_spec(bm, f_in), _full_spec(w.shape),
                  _row_spec(bm, 1), _row_spec(bm, 1)],
        out_specs=_row_spec(bm, h),
        out_shape=jax.ShapeDtypeStruct((n, h), jnp.float32),
    )(x, w, da, db)


def _tc_mid_call(za, zb, y, da, db, b, g, be, m, v, w, bm=2000):
    n, h = y.shape
    h2 = w.shape[1]
    row1 = lambda a: a.reshape(1, -1)
    return pl.pallas_call(
        _tc_mid,
        grid=(n // bm,),
        in_specs=[_row_spec(bm, h)] * 3 + [_row_spec(bm, 1)] * 2
                 + [_full_spec((1, h))] * 5 + [_full_spec(w.shape)],
        out_specs=_row_spec(bm, h2),
        out_shape=jax.ShapeDtypeStruct((n, h2), jnp.float32),
    )(za, zb, y, da, db, row1(b), row1(g), row1(be), row1(m), row1(v), w)


def _tc_last_call(za, zb, y, da, db, b, bm=2000):
    n, c = y.shape
    return pl.pallas_call(
        _tc_last,
        grid=(n // bm,),
        in_specs=[_row_spec(bm, c)] * 3 + [_row_spec(bm, 1)] * 2
                 + [_full_spec((1, c))],
        out_specs=_row_spec(bm, c),
        out_shape=jax.ShapeDtypeStruct((n, c), jnp.float32),
    )(za, zb, y, da, db, b.reshape(1, -1))


def kernel(x, edge_index, W1, b1, g1, be1, m1, v1, W2, b2, g2, be2, m2, v2,
           W3, b3):
    n = x.shape[0]
    e = edge_index.shape[1]
    nw = NC * NS
    rpt = -(-(n + 1) // (NS * LANE)) * LANE
    npad = NS * rpt
    rpw = -(-(-(-e // (nw * LANE))) // 16) * 16
    e_pad = nw * rpw * LANE
    src2d = jnp.concatenate(
        [edge_index[0], jnp.zeros((e_pad - e,), jnp.int32)]).reshape(-1, LANE)
    dst2d = jnp.concatenate(
        [edge_index[1], jnp.full((e_pad - e,), n, jnp.int32)]).reshape(-1, LANE)

    degp = _deg_call(dst2d, npad, rpw, rpt)
    da = degp[0, :n, None]
    db = degp[1, :n, None]

    y1 = _tc_first_call(x, W1, da, db)
    z1 = _agg_call(y1, src2d, dst2d, npad, rpw, rpt)
    y2 = _tc_mid_call(z1[0, :n], z1[1, :n], y1, da, db, b1, g1, be1, m1, v1, W2)
    z2 = _agg_call(y2, src2d, dst2d, npad, rpw, rpt)
    y3 = _tc_mid_call(z2[0, :n], z2[1, :n], y2, da, db, b2, g2, be2, m2, v2, W3)
    z3 = _agg_call(y3, src2d, dst2d, npad, rpw, rpt)
    return _tc_last_call(z3[0, :n], z3[1, :n], y3, da, db, b3)


# speedup vs baseline: 15.1614x; 1.0907x over previous
import functools

import jax
import jax.numpy as jnp
from jax import lax
from jax.experimental import pallas as pl
from jax.experimental.pallas import tpu as pltpu
from jax.experimental.pallas import tpu_sc as plsc

NC = 2
NS = 16
LANE = 128

_MESH = plsc.VectorSubcoreMesh(core_axis_name="c", subcore_axis_name="s")
_SC_PARAMS = pltpu.CompilerParams(use_tc_tiling_on_sc=False)


K0 = 3
K1 = 1


def _core_base(c, s, gpre):
    return jnp.where(c == 0, s * (K0 * gpre),
                     NS * K0 * gpre + s * (K1 * gpre))


def _deg_body(gpre, rpt, nbuf, dst_hbm, ones_hbm, zeros_hbm, out_hbm,
              acc, dst_all, ones_v, buf_v, *sems):
    c = lax.axis_index("c")
    s = lax.axis_index("s")
    pltpu.sync_copy(zeros_hbm, buf_v)
    pltpu.sync_copy(buf_v, acc.at[pl.ds(s * rpt, rpt)])
    pltpu.sync_copy(ones_hbm, ones_v)
    plsc.subcore_barrier()

    @pl.when(c == 0)
    def _():
        pltpu.sync_copy(dst_hbm.at[pl.ds(s * (K0 * gpre), K0 * gpre)],
                        dst_all)

    @pl.when(c != 0)
    def _():
        pltpu.sync_copy(
            dst_hbm.at[pl.ds(NS * K0 * gpre + s * (K1 * gpre), K1 * gpre)],
            dst_all.at[pl.ds(0, K1 * gpre)])

    ngroups = jnp.where(c == 0, (K0 * gpre) // nbuf, (K1 * gpre) // nbuf)
    for b in range(nbuf):
        pltpu.async_copy(ones_v, acc.at[dst_all.at[b]], sems[b], add=True)

    def group(g, carry):
        for b in range(nbuf):
            pltpu.make_async_copy(ones_v, acc.at[dst_all.at[0]],
                                  sems[b]).wait()
            pltpu.async_copy(ones_v, acc.at[dst_all.at[(g + 1) * nbuf + b]],
                             sems[b], add=True)
        return carry

    lax.fori_loop(0, ngroups - 1, group, 0)
    for b in range(nbuf):
        pltpu.make_async_copy(ones_v, acc.at[dst_all.at[0]], sems[b]).wait()
    plsc.subcore_barrier()
    pltpu.sync_copy(acc.at[pl.ds(s * rpt, rpt)], buf_v)
    pltpu.sync_copy(buf_v, out_hbm.at[c, pl.ds(s * rpt, rpt)])


def _agg_body(gpre, rpt, nbuf, y_hbm, src_hbm, dst_hbm, zeros_hbm,
              out_hbm, acc, src_all, dst_all, rows, *sems):
    g_sems = sems[:nbuf]
    s_sems = sems[nbuf:]
    c = lax.axis_index("c")
    s = lax.axis_index("s")
    ngroups = gpre // nbuf
    pltpu.sync_copy(zeros_hbm, rows.at[0])
    for k in range(rpt // LANE):
        pltpu.sync_copy(rows.at[0], acc.at[pl.ds(s * rpt + k * LANE, LANE)])
    plsc.subcore_barrier()
    base = _core_base(c, s, gpre)
    nphases = jnp.where(c == 0, K0, K1)

    def gather(row_in_phase, b):
        return pltpu.async_copy(y_hbm.at[src_all.at[row_in_phase]],
                                rows.at[b], g_sems[b])

    def scatter(row_in_phase, b):
        return pltpu.async_copy(rows.at[b], acc.at[dst_all.at[row_in_phase]],
                                s_sems[b], add=True)

    def phase(ph, carry):
        pbase = base + ph * gpre
        pltpu.sync_copy(src_hbm.at[pl.ds(pbase, gpre)], src_all)
        pltpu.sync_copy(dst_hbm.at[pl.ds(pbase, gpre)], dst_all)
        for b in range(nbuf):
            gather(b, b)

        def group(g, cc):
            for b in range(nbuf):
                pltpu.make_async_copy(y_hbm.at[src_all.at[0]], rows.at[b],
                                      g_sems[b]).wait()
                scatter(g * nbuf + b, b)
            for b in range(nbuf):
                pltpu.make_async_copy(rows.at[b], acc.at[dst_all.at[0]],
                                      s_sems[b]).wait()
                gather((g + 1) * nbuf + b, b)
            return cc

        lax.fori_loop(0, ngroups - 1, group, 0)
        for b in range(nbuf):
            pltpu.make_async_copy(y_hbm.at[src_all.at[0]], rows.at[b],
                                  g_sems[b]).wait()
            scatter((ngroups - 1) * nbuf + b, b)
        for b in range(nbuf):
            pltpu.make_async_copy(rows.at[b], acc.at[dst_all.at[0]],
                                  s_sems[b]).wait()
        return carry

    lax.fori_loop(0, nphases, phase, 0)
    plsc.subcore_barrier()
    for k in range(rpt // LANE):
        r = s * rpt + k * LANE
        pltpu.sync_copy(acc.at[pl.ds(r, LANE)], rows.at[0])
        pltpu.sync_copy(rows.at[0], out_hbm.at[c, pl.ds(r, LANE)])


def _deg_call(dst2d, npad, gpre, rpt, nbuf=8):
    return pl.kernel(
        functools.partial(_deg_body, gpre, rpt, nbuf),
        out_type=jax.ShapeDtypeStruct((NC, npad), jnp.float32),
        mesh=_MESH,
        compiler_params=_SC_PARAMS,
        scratch_types=[
            pltpu.VMEM_SHARED((npad,), jnp.float32),
            pltpu.VMEM((K0 * gpre, LANE), jnp.int32),
            pltpu.VMEM((LANE,), jnp.float32),
            pltpu.VMEM((rpt,), jnp.float32),
        ] + [pltpu.SemaphoreType.DMA] * nbuf,
    )(dst2d, jnp.ones((LANE,), jnp.float32), jnp.zeros((rpt,), jnp.float32))


def _agg_call(y, src2d, dst2d, npad, gpre, rpt):
    f = y.shape[1]
    nbuf = max(2, min(8, 256 // f))
    return pl.kernel(
        functools.partial(_agg_body, gpre, rpt, nbuf),
        out_type=jax.ShapeDtypeStruct((NC, npad, f), jnp.float32),
        mesh=_MESH,
        compiler_params=_SC_PARAMS,
        scratch_types=[
            pltpu.VMEM_SHARED((npad, f), jnp.float32),
            pltpu.VMEM((gpre, LANE), jnp.int32),
            pltpu.VMEM((gpre, LANE), jnp.int32),
            pltpu.VMEM((nbuf, LANE, f), jnp.float32),
        ] + [pltpu.SemaphoreType.DMA] * (2 * nbuf),
    )(y, src2d, dst2d, jnp.zeros((LANE, f), jnp.float32))


def _dot(a, b):
    return jax.lax.dot(a, b, precision=jax.lax.Precision.HIGHEST,
                       preferred_element_type=jnp.float32)


def _tc_first(x_ref, w_ref, da_ref, db_ref, o_ref):
    dinv = lax.rsqrt(da_ref[...] + db_ref[...] + 1.0)
    o_ref[...] = _dot(x_ref[...], w_ref[...]) * dinv


def _tc_mid(za_ref, zb_ref, y_ref, da_ref, db_ref, b_ref, g_ref, be_ref,
            m_ref, v_ref, w_ref, o_ref):
    dinv = lax.rsqrt(da_ref[...] + db_ref[...] + 1.0)
    t = dinv * (za_ref[...] + zb_ref[...] + y_ref[...]) + b_ref[...]
    t = g_ref[...] * (t - m_ref[...]) * lax.rsqrt(v_ref[...] + 1e-5) + be_ref[...]
    t = jnp.maximum(t, 0.0)
    o_ref[...] = _dot(t, w_ref[...]) * dinv


def _tc_last(za_ref, zb_ref, y_ref, da_ref, db_ref, b_ref, o_ref):
    dinv = lax.rsqrt(da_ref[...] + db_ref[...] + 1.0)
    logits = dinv * (za_ref[...] + zb_ref[...] + y_ref[...]) + b_ref[...]
    mx = jnp.max(logits, axis=1, keepdims=True)
    sh = logits - mx
    o_ref[...] = sh - jnp.log(jnp.sum(jnp.exp(sh), axis=1, keepdims=True))


def _row_spec(bm, f):
    return pl.BlockSpec((bm, f), lambda i: (i, 0))


def _full_spec(shape):
    return pl.BlockSpec(shape, lambda i: (0,) * len(shape))


def _tc_first_call(x, w, da, db, bm=2000):
    n, f_in = x.shape
    h = w.shape[1]
    return pl.pallas_call(
        _tc_first,
        grid=(n // bm,),
        in_specs=[_row_spec(bm, f_in), _full_spec(w.shape),
                  _row_spec(bm, 1), _row_spec(bm, 1)],
        out_specs=_row_spec(bm, h),
        out_shape=jax.ShapeDtypeStruct((n, h), jnp.float32),
    )(x, w, da, db)


def _tc_mid_call(za, zb, y, da, db, b, g, be, m, v, w, bm=2000):
    n, h = y.shape
    h2 = w.shape[1]
    row1 = lambda a: a.reshape(1, -1)
    return pl.pallas_call(
        _tc_mid,
        grid=(n // bm,),
        in_specs=[_row_spec(bm, h)] * 3 + [_row_spec(bm, 1)] * 2
                 + [_full_spec((1, h))] * 5 + [_full_spec(w.shape)],
        out_specs=_row_spec(bm, h2),
        out_shape=jax.ShapeDtypeStruct((n, h2), jnp.float32),
    )(za, zb, y, da, db, row1(b), row1(g), row1(be), row1(m), row1(v), w)


def _tc_last_call(za, zb, y, da, db, b, bm=2000):
    n, c = y.shape
    return pl.pallas_call(
        _tc_last,
        grid=(n // bm,),
        in_specs=[_row_spec(bm, c)] * 3 + [_row_spec(bm, 1)] * 2
                 + [_full_spec((1, c))],
        out_specs=_row_spec(bm, c),
        out_shape=jax.ShapeDtypeStruct((n, c), jnp.float32),
    )(za, zb, y, da, db, b.reshape(1, -1))


def kernel(x, edge_index, W1, b1, g1, be1, m1, v1, W2, b2, g2, be2, m2, v2,
           W3, b3):
    n = x.shape[0]
    e = edge_index.shape[1]
    nw = NC * NS
    rpt = -(-(n + 1) // (NS * LANE)) * LANE
    npad = NS * rpt
    gpre = -(-(-(-e // LANE)) // (NS * (K0 + K1) * 8)) * 8
    e_pad = NS * (K0 + K1) * gpre * LANE
    src2d = jnp.concatenate(
        [edge_index[0], jnp.zeros((e_pad - e,), jnp.int32)]).reshape(-1, LANE)
    dst2d = jnp.concatenate(
        [edge_index[1], jnp.full((e_pad - e,), n, jnp.int32)]).reshape(-1, LANE)

    degp = _deg_call(dst2d, npad, gpre, rpt)
    da = degp[0, :n, None]
    db = degp[1, :n, None]

    y1 = _tc_first_call(x, W1, da, db)
    z1 = _agg_call(y1, src2d, dst2d, npad, gpre, rpt)
    y2 = _tc_mid_call(z1[0, :n], z1[1, :n], y1, da, db, b1, g1, be1, m1, v1, W2)
    z2 = _agg_call(y2, src2d, dst2d, npad, gpre, rpt)
    y3 = _tc_mid_call(z2[0, :n], z2[1, :n], y2, da, db, b2, g2, be2, m2, v2, W3)
    z3 = _agg_call(y3, src2d, dst2d, npad, gpre, rpt)
    return _tc_last_call(z3[0, :n], z3[1, :n], y3, da, db, b3)
